# baseline (speedup 1.0000x reference)
import functools

import jax
import jax.numpy as jnp
from jax import lax
from jax.experimental import pallas as pl
from jax.experimental.pallas import tpu as pltpu
from jax.experimental.pallas import tpu_sc as plsc

BATCH_SIZE = 16384
DIM = 64
NUM_CORES = 2
NUM_SUBCORES = 16
NUM_WORKERS = NUM_CORES * NUM_SUBCORES
BPW = BATCH_SIZE // NUM_WORKERS
IDX_CHUNK = 128
N_CHUNKS = BPW // IDX_CHUNK


def _gather_body(uidx, pidx, nidx, uemb, iemb, out_u, out_p, out_n,
                 idx_u, idx_p, idx_n, ru, rp, rn, sem):
    wid = lax.axis_index("s") * NUM_CORES + lax.axis_index("c")
    base = wid * BPW
    pltpu.sync_copy(uidx.at[wid], idx_u)
    pltpu.sync_copy(pidx.at[wid], idx_p)
    pltpu.sync_copy(nidx.at[wid], idx_n)
    handles = []
    for j in range(N_CHUNKS):
        sl = pl.ds(j * IDX_CHUNK, IDX_CHUNK)
        handles.append(pltpu.async_copy(uemb.at[idx_u.at[j]], ru.at[sl], sem))
        handles.append(pltpu.async_copy(iemb.at[idx_p.at[j]], rp.at[sl], sem))
        handles.append(pltpu.async_copy(iemb.at[idx_n.at[j]], rn.at[sl], sem))
    for h in handles:
        h.wait()
    pltpu.sync_copy(ru, out_u.at[pl.ds(base, BPW)])
    pltpu.sync_copy(rp, out_p.at[pl.ds(base, BPW)])
    pltpu.sync_copy(rn, out_n.at[pl.ds(base, BPW)])


_ROWS = jax.ShapeDtypeStruct((BATCH_SIZE, DIM), jnp.float32)


@functools.cache
def _sc_gather():
    return functools.partial(
        pl.kernel,
        mesh=plsc.VectorSubcoreMesh(core_axis_name="c", subcore_axis_name="s"),
        out_type=(_ROWS, _ROWS, _ROWS),
        scratch_types=[
            pltpu.VMEM((N_CHUNKS, IDX_CHUNK), jnp.int32),
            pltpu.VMEM((N_CHUNKS, IDX_CHUNK), jnp.int32),
            pltpu.VMEM((N_CHUNKS, IDX_CHUNK), jnp.int32),
            pltpu.VMEM((BPW, DIM), jnp.float32),
            pltpu.VMEM((BPW, DIM), jnp.float32),
            pltpu.VMEM((BPW, DIM), jnp.float32),
            pltpu.SemaphoreType.DMA,
        ],
        compiler_params=pltpu.CompilerParams(use_tc_tiling_on_sc=False),
    )(_gather_body)


_TC_BLOCK = 2048
_TC_GRID = BATCH_SIZE // _TC_BLOCK


def _loss_body(u_ref, p_ref, n_ref, o_ref):
    i = pl.program_id(0)
    u = u_ref[...]
    up = jnp.sum(u * p_ref[...], axis=1)
    un = jnp.sum(u * n_ref[...], axis=1)
    d = jax.nn.sigmoid(up) - jax.nn.sigmoid(un)
    s = jnp.sum(-jax.nn.log_sigmoid(d)) * (1.0 / BATCH_SIZE)

    @pl.when(i == 0)
    def _():
        o_ref[0, 0] = s

    @pl.when(i > 0)
    def _():
        o_ref[0, 0] += s


_tc_loss = pl.pallas_call(
    _loss_body,
    grid=(_TC_GRID,),
    in_specs=[pl.BlockSpec((_TC_BLOCK, DIM), lambda i: (i, 0))] * 3,
    out_specs=pl.BlockSpec((1, 1), lambda i: (0, 0),
                           memory_space=pltpu.SMEM),
    out_shape=jax.ShapeDtypeStruct((1, 1), jnp.float32),
)


def kernel(user_indices, pos_item_indices, neg_item_indices,
           user_embedding, item_embedding):
    uidx = user_indices.astype(jnp.int32).reshape(NUM_WORKERS, N_CHUNKS, IDX_CHUNK)
    pidx = pos_item_indices.astype(jnp.int32).reshape(NUM_WORKERS, N_CHUNKS, IDX_CHUNK)
    nidx = neg_item_indices.astype(jnp.int32).reshape(NUM_WORKERS, N_CHUNKS, IDX_CHUNK)
    u_rows, p_rows, n_rows = _sc_gather()(uidx, pidx, nidx,
                                          user_embedding, item_embedding)
    out = _tc_loss(u_rows, p_rows, n_rows)
    return out[0, 0]
